# baseline (speedup 1.0000x reference)
import jax
import jax.numpy as jnp
from jax import lax
from jax.experimental import pallas as pl
from jax.experimental.pallas import tpu as pltpu
from jax.experimental.pallas import tpu_sc as plsc

N_ROWS = 100000
HID = 128
NSEG = 256
NC = 2
NS = 16
NW = NC * NS
NGRP = N_ROWS // 8
WROWS = 3136
BR = 224
NBLK = WROWS // BR
BRH = BR * HID
EMBW = NSEG * HID
SEGW = 16 * HID
NEG = float("-inf")


def _sload(ref, idx):
    return ref[pl.ds(idx, 16)][0]


def _sstore(ref, idx, val, lane0):
    plsc.store_scatter(ref, [jnp.full((16,), idx, dtype=jnp.int32)],
                       jnp.full((16,), val, dtype=jnp.float32), mask=lane0)


def _mesh():
    return plsc.VectorSubcoreMesh(
        core_axis_name="c", subcore_axis_name="s",
        num_cores=NC, num_subcores=NS)


def _worker_id():
    return lax.axis_index("c") * NS + lax.axis_index("s")


def _chunk(w):
    g0 = (NGRP * w) // NW
    g1 = (NGRP * (w + 1)) // NW
    o0 = g0 * 8
    p0 = jnp.minimum(o0, N_ROWS - WROWS)
    return o0, (g1 - g0) * 8, p0


def _xcopy_issue(x_hbm, xbuf, sem, p0, bi):
    par = (bi & 1) * BRH
    pltpu.async_copy(x_hbm.at[pl.ds((p0 + bi * BR) * HID, BRH)],
                     xbuf.at[pl.ds(par, BRH)], sem)


def _xcopy_wait(x_hbm, xbuf, sem, p0, bi):
    par = (bi & 1) * BRH
    pltpu.make_async_copy(x_hbm.at[pl.ds((p0 + bi * BR) * HID, BRH)],
                          xbuf.at[pl.ds(par, BRH)], sem).wait()


def _k1_body(x_hbm, b_hbm, part_hbm, xbuf, bbuf, acc, sem):
    w = _worker_id()
    _, _, p0 = _chunk(w)
    neg16 = jnp.full((16,), NEG, dtype=jnp.float32)

    def init_body(i, _):
        acc[pl.ds(i * 16, 16)] = neg16
        return 0
    lax.fori_loop(0, EMBW // 16, init_body, 0)

    pltpu.sync_copy(b_hbm.at[pl.ds(p0, WROWS)], bbuf.at[pl.ds(0, WROWS)])
    cur0 = _sload(bbuf, 0)

    def grp_body(boff, par):
        def f(gi, carry):
            lr = gi * 16
            cur = carry[0]
            segv = bbuf[pl.ds(boff + lr, 16)]
            chg = jnp.any(segv != cur)

            def fast(carry):
                ms = list(carry[1:])
                for r in range(16):
                    for h in range(8):
                        xv = xbuf[pl.ds(par + (lr + r) * HID + h * 16, 16)]
                        ms[h] = jnp.maximum(ms[h], xv)
                return (carry[0], *ms)

            def slow(carry):
                for r in range(16):
                    cur = carry[0]
                    ms = carry[1:]
                    seg = segv[r]
                    ch = seg != cur

                    @pl.when(ch)
                    def _():
                        for h in range(8):
                            acc[pl.ds(cur * HID + h * 16, 16)] = ms[h]

                    nms = []
                    for h in range(8):
                        xv = xbuf[pl.ds(par + (lr + r) * HID + h * 16, 16)]
                        nms.append(jnp.where(ch, xv, jnp.maximum(ms[h], xv)))
                    carry = (seg, *nms)
                return carry

            return lax.cond(chg, slow, fast, carry)
        return f

    _xcopy_issue(x_hbm, xbuf, sem, p0, 0)

    def blk_body(bi, carry):
        @pl.when(bi + 1 < NBLK)
        def _():
            _xcopy_issue(x_hbm, xbuf, sem, p0, bi + 1)
        _xcopy_wait(x_hbm, xbuf, sem, p0, bi)
        return lax.fori_loop(0, BR // 16, grp_body(bi * BR, (bi & 1) * BRH),
                             carry)

    carry = lax.fori_loop(0, NBLK, blk_body, (cur0,) + (neg16,) * 8)
    cur = carry[0]
    for h in range(8):
        acc[pl.ds(cur * HID + h * 16, 16)] = carry[1 + h]

    pltpu.sync_copy(acc, part_hbm.at[w])


def _k2_body(x_hbm, b_hbm, part_hbm, emb_hbm, fl_hbm, pcnt_hbm,
             xbuf, bbuf, embbuf, cbuf, eb, cntbuf, fbuf, spemb, sem, semc):
    c = lax.axis_index("c")
    sid = lax.axis_index("s")
    w = c * NS + sid
    o0r, nown, p0 = _chunk(w)
    d0 = o0r - p0
    lane0 = lax.iota(jnp.int32, 16) == 0

    o0 = sid * SEGW
    for rnd in range(4):
        for t8 in range(8):
            pltpu.async_copy(part_hbm.at[rnd * 8 + t8, pl.ds(o0, SEGW)],
                             cbuf.at[pl.ds(t8 * SEGW, SEGW)], semc)
        for t8 in range(8):
            pltpu.make_async_copy(part_hbm.at[rnd * 8 + t8, pl.ds(o0, SEGW)],
                                  cbuf.at[pl.ds(t8 * SEGW, SEGW)], semc).wait()

        def red_body(j, _, rnd=rnd):
            for u in range(4):
                sl = pl.ds(j * 64 + u * 16, 16)
                v = cbuf[sl]
                for t8 in range(1, 8):
                    v = jnp.maximum(v, cbuf[pl.ds(t8 * SEGW + j * 64 + u * 16,
                                                  16)])
                if rnd > 0:
                    v = jnp.maximum(v, eb[sl])
                eb[sl] = v
            return 0
        lax.fori_loop(0, SEGW // 64, red_body, 0)

    pltpu.sync_copy(eb, spemb.at[pl.ds(o0, SEGW)])

    @pl.when(c == 0)
    def _():
        pltpu.sync_copy(eb, emb_hbm.at[pl.ds(o0, SEGW)])

    plsc.subcore_barrier()
    pltpu.sync_copy(spemb, embbuf)

    zero16 = jnp.zeros((16,), dtype=jnp.float32)
    for j in range(NSEG // 16):
        cntbuf[pl.ds(j * 16, 16)] = zero16

    pltpu.sync_copy(b_hbm.at[pl.ds(p0, WROWS)], bbuf.at[pl.ds(0, WROWS)])
    cur0 = _sload(bbuf, 0)
    iota16 = lax.iota(jnp.int32, 16)
    lane_eq = [iota16 == r for r in range(16)]
    e0 = [embbuf[pl.ds(cur0 * HID + h * 16, 16)] for h in range(8)]

    def grp_body(boff, par):
        def f(gi, carry):
            lr = gi * 16
            segv = bbuf[pl.ds(boff + lr, 16)]
            chg = jnp.any(segv != carry[0])
            grow = boff + lr + iota16
            validf = jnp.where((grow >= d0) & (grow < d0 + nown), 1.0, 0.0)

            def fast(carry):
                cur, cnt = carry[0], carry[1]
                es = carry[2:]
                fv = jnp.zeros((16,), dtype=jnp.float32)
                for r in range(16):
                    anyv = jnp.zeros((16,), dtype=jnp.bool_)
                    for h in range(8):
                        xv = xbuf[pl.ds(par + (lr + r) * HID + h * 16, 16)]
                        anyv = anyv | (xv == es[h])
                    pop = plsc.all_reduce_population_count(anyv)
                    fls = jnp.minimum(pop, 1).astype(jnp.float32)
                    fv = jnp.where(lane_eq[r], fls, fv)
                fbuf[pl.ds(lr, 16)] = fv
                cnt = cnt + jnp.sum(fv * validf)
                return (cur, cnt) + tuple(es)

            def slow(carry):
                for r in range(16):
                    cur, cnt = carry[0], carry[1]
                    seg = segv[r]
                    ch = seg != cur

                    @pl.when(ch)
                    def _():
                        _sstore(cntbuf, cur, cnt, lane0)

                    anyv = jnp.zeros((16,), dtype=jnp.bool_)
                    nes = []
                    for h in range(8):
                        xv = xbuf[pl.ds(par + (lr + r) * HID + h * 16, 16)]
                        ev = embbuf[pl.ds(seg * HID + h * 16, 16)]
                        nes.append(ev)
                        anyv = anyv | (xv == ev)
                    fl = jnp.where(jnp.any(anyv), 1.0, 0.0)
                    _sstore(fbuf, lr + r, fl, lane0)
                    g = boff + lr + r
                    valid = (g >= d0) & (g < d0 + nown)
                    cnt = jnp.where(ch, 0.0, cnt) + jnp.where(valid, fl, 0.0)
                    carry = (seg, cnt) + tuple(nes)
                return carry

            return lax.cond(chg, slow, fast, carry)
        return f

    _xcopy_issue(x_hbm, xbuf, sem, p0, 0)

    def blk_body(bi, carry):
        @pl.when(bi + 1 < NBLK)
        def _():
            _xcopy_issue(x_hbm, xbuf, sem, p0, bi + 1)
        _xcopy_wait(x_hbm, xbuf, sem, p0, bi)
        carry = lax.fori_loop(0, BR // 16, grp_body(bi * BR, (bi & 1) * BRH),
                              carry)
        pltpu.sync_copy(fbuf, fl_hbm.at[pl.ds(p0 + bi * BR, BR)])
        return carry

    carry = lax.fori_loop(0, NBLK, blk_body, (cur0, 0.0) + tuple(e0))
    cur, cnt = carry[0], carry[1]
    _sstore(cntbuf, cur, cnt, lane0)
    pltpu.sync_copy(cntbuf, pcnt_hbm.at[pl.ds(w * NSEG, NSEG)])


def _k3_body(b_hbm, fl_hbm, pcnt_hbm, sc_hbm, pbuf, invbuf, bbuf, fbuf, sbuf):
    w = _worker_id()
    _, _, p0 = _chunk(w)

    pltpu.sync_copy(pcnt_hbm, pbuf)
    one16 = jnp.full((16,), 1.0, dtype=jnp.float32)
    for j in range(NSEG // 16):
        tot = pbuf[pl.ds(j * 16, 16)]
        for t in range(1, NW):
            tot = tot + pbuf[pl.ds(t * NSEG + j * 16, 16)]
        invbuf[pl.ds(j * 16, 16)] = one16 / jnp.maximum(tot, one16)

    pltpu.sync_copy(b_hbm.at[pl.ds(p0, WROWS)], bbuf.at[pl.ds(0, WROWS)])
    pltpu.sync_copy(fl_hbm.at[pl.ds(p0, WROWS)], fbuf)

    def loop_body(j, _):
        sl = pl.ds(j * 16, 16)
        ids = jnp.clip(bbuf[sl], 0, NSEG - 1)
        iv = plsc.load_gather(invbuf, [ids])
        sbuf[sl] = fbuf[sl] * iv
        return 0
    lax.fori_loop(0, WROWS // 16, loop_body, 0)

    pltpu.sync_copy(sbuf, sc_hbm.at[pl.ds(p0, WROWS)])


BT = 1000
NSTEP = N_ROWS // BT


def _k2tc_body(part_ref, x_ref, b_ref, emb_ref, fl_ref, tot_ref, emb_s, tot_s):
    pid = pl.program_id(0)

    @pl.when(pid == 0)
    def _():
        emb_s[...] = jnp.max(part_ref[...], axis=0)
        emb_ref[...] = emb_s[...]
        tot_s[...] = jnp.zeros_like(tot_s)

    ids = b_ref[0, 0, :]
    oh = (lax.broadcasted_iota(jnp.int32, (BT, NSEG), 1)
          == ids[:, None]).astype(jnp.float32)
    sel = jnp.dot(oh, emb_s[...], precision=lax.Precision.HIGHEST,
                  preferred_element_type=jnp.float32)
    flags = jnp.any(x_ref[...] == sel, axis=1).astype(jnp.float32)
    fl_ref[0, 0, :] = flags
    oh_t = (lax.broadcasted_iota(jnp.int32, (NSEG, BT), 0)
            == ids[None, :]).astype(jnp.float32)
    tot_s[...] = tot_s[...] + jnp.dot(
        oh_t, flags, precision=lax.Precision.HIGHEST,
        preferred_element_type=jnp.float32).reshape(1, NSEG)

    @pl.when(pid == NSTEP - 1)
    def _():
        tot_ref[...] = tot_s[...]


def _k3tc_body(tot_ref, b_ref, fl_ref, sc_ref):
    inv = 1.0 / jnp.maximum(tot_ref[...], 1.0)
    ids = b_ref[0, 0, :]
    oh = (lax.broadcasted_iota(jnp.int32, (BT, NSEG), 1)
          == ids[:, None]).astype(jnp.float32)
    selinv = jnp.dot(oh, inv[0, :], precision=lax.Precision.HIGHEST,
                     preferred_element_type=jnp.float32)
    sc_ref[0, 0, :] = fl_ref[0, 0, :] * selinv


_k2tc = pl.pallas_call(
    _k2tc_body,
    grid=(NSTEP,),
    in_specs=[
        pl.BlockSpec((NW, NSEG, HID), lambda i: (0, 0, 0)),
        pl.BlockSpec((BT, HID), lambda i: (i, 0)),
        pl.BlockSpec((1, 1, BT), lambda i: (i, 0, 0)),
    ],
    out_specs=[
        pl.BlockSpec((NSEG, HID), lambda i: (0, 0)),
        pl.BlockSpec((1, 1, BT), lambda i: (i, 0, 0)),
        pl.BlockSpec((1, NSEG), lambda i: (0, 0)),
    ],
    out_shape=[
        jax.ShapeDtypeStruct((NSEG, HID), jnp.float32),
        jax.ShapeDtypeStruct((NSTEP, 1, BT), jnp.float32),
        jax.ShapeDtypeStruct((1, NSEG), jnp.float32),
    ],
    scratch_shapes=[
        pltpu.VMEM((NSEG, HID), jnp.float32),
        pltpu.VMEM((1, NSEG), jnp.float32),
    ],
)

_k3tc = pl.pallas_call(
    _k3tc_body,
    grid=(NSTEP,),
    in_specs=[
        pl.BlockSpec((1, NSEG), lambda i: (0, 0)),
        pl.BlockSpec((1, 1, BT), lambda i: (i, 0, 0)),
        pl.BlockSpec((1, 1, BT), lambda i: (i, 0, 0)),
    ],
    out_specs=pl.BlockSpec((1, 1, BT), lambda i: (i, 0, 0)),
    out_shape=jax.ShapeDtypeStruct((NSTEP, 1, BT), jnp.float32),
)


_k1 = pl.kernel(
    _k1_body,
    out_type=jax.ShapeDtypeStruct((NW, EMBW), jnp.float32),
    mesh=_mesh(),
    compiler_params=pltpu.CompilerParams(needs_layout_passes=False),
    scratch_types=[
        pltpu.VMEM((2 * BRH,), jnp.float32),
        pltpu.VMEM((WROWS + 16,), jnp.int32),
        pltpu.VMEM((EMBW,), jnp.float32),
        pltpu.SemaphoreType.DMA,
    ],
)

_k2 = pl.kernel(
    _k2_body,
    out_type=(
        jax.ShapeDtypeStruct((EMBW,), jnp.float32),
        jax.ShapeDtypeStruct((N_ROWS,), jnp.float32),
        jax.ShapeDtypeStruct((NW * NSEG,), jnp.float32),
    ),
    mesh=_mesh(),
    compiler_params=pltpu.CompilerParams(needs_layout_passes=False),
    scratch_types=[
        pltpu.VMEM((2 * BRH,), jnp.float32),
        pltpu.VMEM((WROWS + 16,), jnp.int32),
        pltpu.VMEM((EMBW,), jnp.float32),
        pltpu.VMEM((8 * SEGW,), jnp.float32),
        pltpu.VMEM((SEGW,), jnp.float32),
        pltpu.VMEM((NSEG,), jnp.float32),
        pltpu.VMEM((BR,), jnp.float32),
        pltpu.VMEM_SHARED((EMBW,), jnp.float32),
        pltpu.SemaphoreType.DMA,
        pltpu.SemaphoreType.DMA,
    ],
)

_k3 = pl.kernel(
    _k3_body,
    out_type=jax.ShapeDtypeStruct((N_ROWS,), jnp.float32),
    mesh=_mesh(),
    compiler_params=pltpu.CompilerParams(needs_layout_passes=False),
    scratch_types=[
        pltpu.VMEM((NW * NSEG,), jnp.float32),
        pltpu.VMEM((NSEG,), jnp.float32),
        pltpu.VMEM((WROWS + 16,), jnp.int32),
        pltpu.VMEM((WROWS,), jnp.float32),
        pltpu.VMEM((WROWS,), jnp.float32),
    ],
)


def kernel(x, batch):
    xf = x.reshape(-1)
    part = _k1(xf, batch)
    b3 = batch.reshape(NSTEP, 1, BT)
    emb, fl3, tot = _k2tc(part.reshape(NW, NSEG, HID), x, b3)
    sc3 = _k3tc(tot, b3, fl3)
    return emb, sc3.reshape(N_ROWS)


# speedup vs baseline: 3.3605x; 3.3605x over previous
import jax
import jax.numpy as jnp
from jax import lax
from jax.experimental import pallas as pl
from jax.experimental.pallas import tpu as pltpu
from jax.experimental.pallas import tpu_sc as plsc

N_ROWS = 100000
HID = 128
NSEG = 256
NC = 2
NS = 16
NW = NC * NS
NGRP = N_ROWS // 8
WROWS = 3136
BR = 224
NBLK = WROWS // BR
BRH = BR * HID
EMBW = NSEG * HID
SEGW = 16 * HID
NEG = float("-inf")


def _sload(ref, idx):
    return ref[pl.ds(idx, 16)][0]


def _sstore(ref, idx, val, lane0):
    plsc.store_scatter(ref, [jnp.full((16,), idx, dtype=jnp.int32)],
                       jnp.full((16,), val, dtype=jnp.float32), mask=lane0)


def _mesh():
    return plsc.VectorSubcoreMesh(
        core_axis_name="c", subcore_axis_name="s",
        num_cores=NC, num_subcores=NS)


def _worker_id():
    return lax.axis_index("c") * NS + lax.axis_index("s")


def _chunk(w):
    g0 = (NGRP * w) // NW
    g1 = (NGRP * (w + 1)) // NW
    o0 = g0 * 8
    p0 = jnp.minimum(o0, N_ROWS - WROWS)
    return o0, (g1 - g0) * 8, p0


def _xcopy_issue(x_hbm, xbuf, sem, p0, bi):
    par = (bi & 1) * BRH
    pltpu.async_copy(x_hbm.at[pl.ds((p0 + bi * BR) * HID, BRH)],
                     xbuf.at[pl.ds(par, BRH)], sem)


def _xcopy_wait(x_hbm, xbuf, sem, p0, bi):
    par = (bi & 1) * BRH
    pltpu.make_async_copy(x_hbm.at[pl.ds((p0 + bi * BR) * HID, BRH)],
                          xbuf.at[pl.ds(par, BRH)], sem).wait()


def _k1_body(x_hbm, b_hbm, part_hbm, xbuf, bbuf, acc, sem, semb):
    w = _worker_id()
    _, _, p0 = _chunk(w)
    neg16 = jnp.full((16,), NEG, dtype=jnp.float32)

    _xcopy_issue(x_hbm, xbuf, sem, p0, 0)
    bcopy = pltpu.make_async_copy(b_hbm.at[pl.ds(p0, WROWS)],
                                  bbuf.at[pl.ds(0, WROWS)], semb)
    bcopy.start()

    def init_body(i, _):
        for u in range(16):
            acc[pl.ds(i * 256 + u * 16, 16)] = neg16
        return 0
    lax.fori_loop(0, EMBW // 256, init_body, 0)

    bcopy.wait()
    cur0 = _sload(bbuf, 0)

    def grp_body(boff, par):
        def f(gi, carry):
            lr = gi * 16
            cur = carry[0]
            segv = bbuf[pl.ds(boff + lr, 16)]
            chg = jnp.any(segv != cur)

            def fast(carry):
                ms = list(carry[1:])
                for r in range(16):
                    for h in range(8):
                        xv = xbuf[pl.ds(par + (lr + r) * HID + h * 16, 16)]
                        ms[h] = jnp.maximum(ms[h], xv)
                return (carry[0], *ms)

            def slow(carry):
                for r in range(16):
                    cur = carry[0]
                    ms = carry[1:]
                    seg = segv[r]
                    ch = seg != cur

                    @pl.when(ch)
                    def _():
                        for h in range(8):
                            acc[pl.ds(cur * HID + h * 16, 16)] = ms[h]

                    nms = []
                    for h in range(8):
                        xv = xbuf[pl.ds(par + (lr + r) * HID + h * 16, 16)]
                        nms.append(jnp.where(ch, xv, jnp.maximum(ms[h], xv)))
                    carry = (seg, *nms)
                return carry

            return lax.cond(chg, slow, fast, carry)
        return f

    def blk_body(bi, carry):
        @pl.when(bi + 1 < NBLK)
        def _():
            _xcopy_issue(x_hbm, xbuf, sem, p0, bi + 1)
        _xcopy_wait(x_hbm, xbuf, sem, p0, bi)
        return lax.fori_loop(0, BR // 16, grp_body(bi * BR, (bi & 1) * BRH),
                             carry)

    carry = lax.fori_loop(0, NBLK, blk_body, (cur0,) + (neg16,) * 8)
    cur = carry[0]
    for h in range(8):
        acc[pl.ds(cur * HID + h * 16, 16)] = carry[1 + h]

    pltpu.sync_copy(acc, part_hbm.at[w])


def _k2_body(x_hbm, b_hbm, part_hbm, emb_hbm, fl_hbm, pcnt_hbm,
             xbuf, bbuf, embbuf, cbuf, eb, cntbuf, fbuf, spemb, sem, semc,
             semb):
    c = lax.axis_index("c")
    sid = lax.axis_index("s")
    w = c * NS + sid
    o0r, nown, p0 = _chunk(w)
    d0 = o0r - p0
    lane0 = lax.iota(jnp.int32, 16) == 0

    _xcopy_issue(x_hbm, xbuf, sem, p0, 0)
    bcopy = pltpu.make_async_copy(b_hbm.at[pl.ds(p0, WROWS)],
                                  bbuf.at[pl.ds(0, WROWS)], semb)
    bcopy.start()

    o0 = sid * SEGW
    for rnd in range(4):
        for t8 in range(8):
            pltpu.async_copy(part_hbm.at[rnd * 8 + t8, pl.ds(o0, SEGW)],
                             cbuf.at[pl.ds(t8 * SEGW, SEGW)], semc)
        for t8 in range(8):
            pltpu.make_async_copy(part_hbm.at[rnd * 8 + t8, pl.ds(o0, SEGW)],
                                  cbuf.at[pl.ds(t8 * SEGW, SEGW)], semc).wait()

        def red_body(j, _, rnd=rnd):
            for u in range(4):
                sl = pl.ds(j * 64 + u * 16, 16)
                v = cbuf[sl]
                for t8 in range(1, 8):
                    v = jnp.maximum(v, cbuf[pl.ds(t8 * SEGW + j * 64 + u * 16,
                                                  16)])
                if rnd > 0:
                    v = jnp.maximum(v, eb[sl])
                eb[sl] = v
            return 0
        lax.fori_loop(0, SEGW // 64, red_body, 0)

    pltpu.sync_copy(eb, spemb.at[pl.ds(o0, SEGW)])

    @pl.when(c == 0)
    def _():
        pltpu.sync_copy(eb, emb_hbm.at[pl.ds(o0, SEGW)])

    plsc.subcore_barrier()
    pltpu.sync_copy(spemb, embbuf)

    zero16 = jnp.zeros((16,), dtype=jnp.float32)
    for j in range(NSEG // 16):
        cntbuf[pl.ds(j * 16, 16)] = zero16

    bcopy.wait()
    cur0 = _sload(bbuf, 0)
    iota16 = lax.iota(jnp.int32, 16)
    lane_eq = [iota16 == r for r in range(16)]
    e0 = [embbuf[pl.ds(cur0 * HID + h * 16, 16)] for h in range(8)]

    def grp_body(boff, par):
        def f(gi, carry):
            lr = gi * 16
            segv = bbuf[pl.ds(boff + lr, 16)]
            chg = jnp.any(segv != carry[0])
            grow = boff + lr + iota16
            validf = jnp.where((grow >= d0) & (grow < d0 + nown), 1.0, 0.0)

            def fast(carry):
                cur, cnt = carry[0], carry[1]
                es = carry[2:]
                fv = jnp.zeros((16,), dtype=jnp.float32)
                for r in range(16):
                    anyv = jnp.zeros((16,), dtype=jnp.bool_)
                    for h in range(8):
                        xv = xbuf[pl.ds(par + (lr + r) * HID + h * 16, 16)]
                        anyv = anyv | (xv == es[h])
                    pop = plsc.all_reduce_population_count(anyv)
                    fls = jnp.minimum(pop, 1).astype(jnp.float32)
                    fv = jnp.where(lane_eq[r], fls, fv)
                fbuf[pl.ds(lr, 16)] = fv
                cnt = cnt + jnp.sum(fv * validf)
                return (cur, cnt) + tuple(es)

            def slow(carry):
                for r in range(16):
                    cur, cnt = carry[0], carry[1]
                    seg = segv[r]
                    ch = seg != cur

                    @pl.when(ch)
                    def _():
                        _sstore(cntbuf, cur, cnt, lane0)

                    anyv = jnp.zeros((16,), dtype=jnp.bool_)
                    nes = []
                    for h in range(8):
                        xv = xbuf[pl.ds(par + (lr + r) * HID + h * 16, 16)]
                        ev = embbuf[pl.ds(seg * HID + h * 16, 16)]
                        nes.append(ev)
                        anyv = anyv | (xv == ev)
                    fl = jnp.where(jnp.any(anyv), 1.0, 0.0)
                    _sstore(fbuf, lr + r, fl, lane0)
                    g = boff + lr + r
                    valid = (g >= d0) & (g < d0 + nown)
                    cnt = jnp.where(ch, 0.0, cnt) + jnp.where(valid, fl, 0.0)
                    carry = (seg, cnt) + tuple(nes)
                return carry

            return lax.cond(chg, slow, fast, carry)
        return f

    def blk_body(bi, carry):
        @pl.when(bi + 1 < NBLK)
        def _():
            _xcopy_issue(x_hbm, xbuf, sem, p0, bi + 1)
        _xcopy_wait(x_hbm, xbuf, sem, p0, bi)
        carry = lax.fori_loop(0, BR // 16, grp_body(bi * BR, (bi & 1) * BRH),
                              carry)
        pltpu.sync_copy(fbuf, fl_hbm.at[pl.ds(p0 + bi * BR, BR)])
        return carry

    carry = lax.fori_loop(0, NBLK, blk_body, (cur0, 0.0) + tuple(e0))
    cur, cnt = carry[0], carry[1]
    _sstore(cntbuf, cur, cnt, lane0)
    pltpu.sync_copy(cntbuf, pcnt_hbm.at[pl.ds(w * NSEG, NSEG)])


def _k3_body(b_hbm, fl_hbm, pcnt_hbm, sc_hbm, pbuf, invbuf, bbuf, fbuf, sbuf):
    w = _worker_id()
    _, _, p0 = _chunk(w)

    pltpu.sync_copy(pcnt_hbm, pbuf)
    one16 = jnp.full((16,), 1.0, dtype=jnp.float32)
    for j in range(NSEG // 16):
        tot = pbuf[pl.ds(j * 16, 16)]
        for t in range(1, NW):
            tot = tot + pbuf[pl.ds(t * NSEG + j * 16, 16)]
        invbuf[pl.ds(j * 16, 16)] = one16 / jnp.maximum(tot, one16)

    pltpu.sync_copy(b_hbm.at[pl.ds(p0, WROWS)], bbuf.at[pl.ds(0, WROWS)])
    pltpu.sync_copy(fl_hbm.at[pl.ds(p0, WROWS)], fbuf)

    def loop_body(j, _):
        sl = pl.ds(j * 16, 16)
        ids = jnp.clip(bbuf[sl], 0, NSEG - 1)
        iv = plsc.load_gather(invbuf, [ids])
        sbuf[sl] = fbuf[sl] * iv
        return 0
    lax.fori_loop(0, WROWS // 16, loop_body, 0)

    pltpu.sync_copy(sbuf, sc_hbm.at[pl.ds(p0, WROWS)])


_k1 = pl.kernel(
    _k1_body,
    out_type=jax.ShapeDtypeStruct((NW, EMBW), jnp.float32),
    mesh=_mesh(),
    compiler_params=pltpu.CompilerParams(needs_layout_passes=False),
    scratch_types=[
        pltpu.VMEM((2 * BRH,), jnp.float32),
        pltpu.VMEM((WROWS + 16,), jnp.int32),
        pltpu.VMEM((EMBW,), jnp.float32),
        pltpu.SemaphoreType.DMA,
        pltpu.SemaphoreType.DMA,
    ],
)

_k2 = pl.kernel(
    _k2_body,
    out_type=(
        jax.ShapeDtypeStruct((EMBW,), jnp.float32),
        jax.ShapeDtypeStruct((N_ROWS,), jnp.float32),
        jax.ShapeDtypeStruct((NW * NSEG,), jnp.float32),
    ),
    mesh=_mesh(),
    compiler_params=pltpu.CompilerParams(needs_layout_passes=False),
    scratch_types=[
        pltpu.VMEM((2 * BRH,), jnp.float32),
        pltpu.VMEM((WROWS + 16,), jnp.int32),
        pltpu.VMEM((EMBW,), jnp.float32),
        pltpu.VMEM((8 * SEGW,), jnp.float32),
        pltpu.VMEM((SEGW,), jnp.float32),
        pltpu.VMEM((NSEG,), jnp.float32),
        pltpu.VMEM((BR,), jnp.float32),
        pltpu.VMEM_SHARED((EMBW,), jnp.float32),
        pltpu.SemaphoreType.DMA,
        pltpu.SemaphoreType.DMA,
        pltpu.SemaphoreType.DMA,
    ],
)

_k3 = pl.kernel(
    _k3_body,
    out_type=jax.ShapeDtypeStruct((N_ROWS,), jnp.float32),
    mesh=_mesh(),
    compiler_params=pltpu.CompilerParams(needs_layout_passes=False),
    scratch_types=[
        pltpu.VMEM((NW * NSEG,), jnp.float32),
        pltpu.VMEM((NSEG,), jnp.float32),
        pltpu.VMEM((WROWS + 16,), jnp.int32),
        pltpu.VMEM((WROWS,), jnp.float32),
        pltpu.VMEM((WROWS,), jnp.float32),
    ],
)


def kernel(x, batch):
    xf = x.reshape(-1)
    part = _k1(xf, batch)
    emb, flags, pcnt = _k2(xf, batch, part)
    scores = _k3(batch, flags, pcnt)
    return emb.reshape(NSEG, HID), scores


# speedup vs baseline: 3.4012x; 1.0121x over previous
import jax
import jax.numpy as jnp
from jax import lax
from jax.experimental import pallas as pl
from jax.experimental.pallas import tpu as pltpu
from jax.experimental.pallas import tpu_sc as plsc

N_ROWS = 100000
HID = 128
NSEG = 256
NC = 2
NS = 16
NW = NC * NS
NGRP = N_ROWS // 8
WROWS = 3136
BR = 224
NBLK = WROWS // BR
BRH = BR * HID
EMBW = NSEG * HID
SEGW = 16 * HID
NEG = float("-inf")


def _sload(ref, idx):
    return ref[pl.ds(idx, 16)][0]


def _sstore(ref, idx, val, lane0):
    plsc.store_scatter(ref, [jnp.full((16,), idx, dtype=jnp.int32)],
                       jnp.full((16,), val, dtype=jnp.float32), mask=lane0)


def _mesh():
    return plsc.VectorSubcoreMesh(
        core_axis_name="c", subcore_axis_name="s",
        num_cores=NC, num_subcores=NS)


def _worker_id():
    return lax.axis_index("c") * NS + lax.axis_index("s")


def _chunk(w):
    g0 = (NGRP * w) // NW
    g1 = (NGRP * (w + 1)) // NW
    o0 = g0 * 8
    p0 = jnp.minimum(o0, N_ROWS - WROWS)
    return o0, (g1 - g0) * 8, p0


def _xcopy_issue(x_hbm, xbuf, sem, p0, bi):
    par = (bi & 1) * BRH
    pltpu.async_copy(x_hbm.at[pl.ds((p0 + bi * BR) * HID, BRH)],
                     xbuf.at[pl.ds(par, BRH)], sem)


def _xcopy_wait(x_hbm, xbuf, sem, p0, bi):
    par = (bi & 1) * BRH
    pltpu.make_async_copy(x_hbm.at[pl.ds((p0 + bi * BR) * HID, BRH)],
                          xbuf.at[pl.ds(par, BRH)], sem).wait()


def _k1_body(x_hbm, b_hbm, part_hbm, xbuf, bbuf, acc, sem, semb):
    w = _worker_id()
    _, _, p0 = _chunk(w)
    neg16 = jnp.full((16,), NEG, dtype=jnp.float32)

    _xcopy_issue(x_hbm, xbuf, sem, p0, 0)
    bcopy = pltpu.make_async_copy(b_hbm.at[pl.ds(p0, WROWS)],
                                  bbuf.at[pl.ds(0, WROWS)], semb)
    bcopy.start()

    def init_body(i, _):
        for u in range(16):
            acc[pl.ds(i * 256 + u * 16, 16)] = neg16
        return 0
    lax.fori_loop(0, EMBW // 256, init_body, 0)

    bcopy.wait()
    cur0 = _sload(bbuf, 0)

    def grp_body(boff, par):
        def f(gi, carry):
            lr = gi * 16
            cur = carry[0]
            segv = bbuf[pl.ds(boff + lr, 16)]
            chg = jnp.any(segv != cur)

            def fast(carry):
                ms = list(carry[1:])
                for r in range(16):
                    for h in range(8):
                        xv = xbuf[pl.ds(par + (lr + r) * HID + h * 16, 16)]
                        ms[h] = jnp.maximum(ms[h], xv)
                return (carry[0], *ms)

            def slow(carry):
                for r in range(16):
                    cur = carry[0]
                    ms = carry[1:]
                    seg = segv[r]
                    ch = seg != cur

                    @pl.when(ch)
                    def _():
                        for h in range(8):
                            acc[pl.ds(cur * HID + h * 16, 16)] = ms[h]

                    nms = []
                    for h in range(8):
                        xv = xbuf[pl.ds(par + (lr + r) * HID + h * 16, 16)]
                        nms.append(jnp.where(ch, xv, jnp.maximum(ms[h], xv)))
                    carry = (seg, *nms)
                return carry

            return lax.cond(chg, slow, fast, carry)
        return f

    def blk_body(bi, carry):
        @pl.when(bi + 1 < NBLK)
        def _():
            _xcopy_issue(x_hbm, xbuf, sem, p0, bi + 1)
        _xcopy_wait(x_hbm, xbuf, sem, p0, bi)
        return lax.fori_loop(0, BR // 16, grp_body(bi * BR, (bi & 1) * BRH),
                             carry)

    carry = lax.fori_loop(0, NBLK, blk_body, (cur0,) + (neg16,) * 8)
    cur = carry[0]
    for h in range(8):
        acc[pl.ds(cur * HID + h * 16, 16)] = carry[1 + h]

    pltpu.sync_copy(acc, part_hbm.at[w])


def _k2_body(x_hbm, b_hbm, part_hbm, emb_hbm, fl_hbm, pcnt_hbm,
             xbuf, bbuf, embbuf, cbuf, eb, cntbuf, fbuf, spemb, sem, semc,
             semb):
    c = lax.axis_index("c")
    sid = lax.axis_index("s")
    w = c * NS + sid
    o0r, nown, p0 = _chunk(w)
    d0 = o0r - p0
    lane0 = lax.iota(jnp.int32, 16) == 0

    _xcopy_issue(x_hbm, xbuf, sem, p0, 0)
    bcopy = pltpu.make_async_copy(b_hbm.at[pl.ds(p0, WROWS)],
                                  bbuf.at[pl.ds(0, WROWS)], semb)
    bcopy.start()

    o0 = sid * SEGW
    for rnd in range(4):
        for t8 in range(8):
            pltpu.async_copy(part_hbm.at[rnd * 8 + t8, pl.ds(o0, SEGW)],
                             cbuf.at[pl.ds(t8 * SEGW, SEGW)], semc)
        for t8 in range(8):
            pltpu.make_async_copy(part_hbm.at[rnd * 8 + t8, pl.ds(o0, SEGW)],
                                  cbuf.at[pl.ds(t8 * SEGW, SEGW)], semc).wait()

        def red_body(j, _, rnd=rnd):
            for u in range(4):
                sl = pl.ds(j * 64 + u * 16, 16)
                v = cbuf[sl]
                for t8 in range(1, 8):
                    v = jnp.maximum(v, cbuf[pl.ds(t8 * SEGW + j * 64 + u * 16,
                                                  16)])
                if rnd > 0:
                    v = jnp.maximum(v, eb[sl])
                eb[sl] = v
            return 0
        lax.fori_loop(0, SEGW // 64, red_body, 0)

    pltpu.sync_copy(eb, spemb.at[pl.ds(o0, SEGW)])

    @pl.when(c == 0)
    def _():
        pltpu.sync_copy(eb, emb_hbm.at[pl.ds(o0, SEGW)])

    plsc.subcore_barrier()

    zero16 = jnp.zeros((16,), dtype=jnp.float32)
    for j in range(NSEG // 16):
        cntbuf[pl.ds(j * 16, 16)] = zero16

    bcopy.wait()
    cur0 = _sload(bbuf, 0)
    seg_hi = _sload(bbuf, WROWS - 1)
    base = jnp.minimum(cur0, NSEG - 32)

    @pl.when(seg_hi - base < 32)
    def _():
        pltpu.sync_copy(spemb.at[pl.ds(base * HID, 32 * HID)],
                        embbuf.at[pl.ds(base * HID, 32 * HID)])

    @pl.when(seg_hi - base >= 32)
    def _():
        pltpu.sync_copy(spemb, embbuf)
    iota16 = lax.iota(jnp.int32, 16)
    lane_eq = [iota16 == r for r in range(16)]
    e0 = [embbuf[pl.ds(cur0 * HID + h * 16, 16)] for h in range(8)]

    def grp_body(boff, par):
        def f(gi, carry):
            lr = gi * 16
            segv = bbuf[pl.ds(boff + lr, 16)]
            chg = jnp.any(segv != carry[0])
            grow = boff + lr + iota16
            validf = jnp.where((grow >= d0) & (grow < d0 + nown), 1.0, 0.0)

            def fast(carry):
                cur, cnt = carry[0], carry[1]
                es = carry[2:]
                fv = jnp.zeros((16,), dtype=jnp.float32)
                for r in range(16):
                    anyv = jnp.zeros((16,), dtype=jnp.bool_)
                    for h in range(8):
                        xv = xbuf[pl.ds(par + (lr + r) * HID + h * 16, 16)]
                        anyv = anyv | (xv == es[h])
                    pop = plsc.all_reduce_population_count(anyv)
                    fls = jnp.minimum(pop, 1).astype(jnp.float32)
                    fv = jnp.where(lane_eq[r], fls, fv)
                fbuf[pl.ds(lr, 16)] = fv
                cnt = cnt + jnp.sum(fv * validf)
                return (cur, cnt) + tuple(es)

            def slow(carry):
                for r in range(16):
                    cur, cnt = carry[0], carry[1]
                    seg = segv[r]
                    ch = seg != cur

                    @pl.when(ch)
                    def _():
                        _sstore(cntbuf, cur, cnt, lane0)

                    anyv = jnp.zeros((16,), dtype=jnp.bool_)
                    nes = []
                    for h in range(8):
                        xv = xbuf[pl.ds(par + (lr + r) * HID + h * 16, 16)]
                        ev = embbuf[pl.ds(seg * HID + h * 16, 16)]
                        nes.append(ev)
                        anyv = anyv | (xv == ev)
                    fl = jnp.where(jnp.any(anyv), 1.0, 0.0)
                    _sstore(fbuf, lr + r, fl, lane0)
                    g = boff + lr + r
                    valid = (g >= d0) & (g < d0 + nown)
                    cnt = jnp.where(ch, 0.0, cnt) + jnp.where(valid, fl, 0.0)
                    carry = (seg, cnt) + tuple(nes)
                return carry

            return lax.cond(chg, slow, fast, carry)
        return f

    def blk_body(bi, carry):
        @pl.when(bi + 1 < NBLK)
        def _():
            _xcopy_issue(x_hbm, xbuf, sem, p0, bi + 1)
        _xcopy_wait(x_hbm, xbuf, sem, p0, bi)
        carry = lax.fori_loop(0, BR // 16, grp_body(bi * BR, (bi & 1) * BRH),
                              carry)
        pltpu.sync_copy(fbuf, fl_hbm.at[pl.ds(p0 + bi * BR, BR)])
        return carry

    carry = lax.fori_loop(0, NBLK, blk_body, (cur0, 0.0) + tuple(e0))
    cur, cnt = carry[0], carry[1]
    _sstore(cntbuf, cur, cnt, lane0)
    pltpu.sync_copy(cntbuf, pcnt_hbm.at[pl.ds(w * NSEG, NSEG)])


def _k3_body(b_hbm, fl_hbm, pcnt_hbm, sc_hbm, pbuf, invbuf, bbuf, fbuf, sbuf):
    w = _worker_id()
    _, _, p0 = _chunk(w)

    pltpu.sync_copy(pcnt_hbm, pbuf)
    one16 = jnp.full((16,), 1.0, dtype=jnp.float32)
    for j in range(NSEG // 16):
        tot = pbuf[pl.ds(j * 16, 16)]
        for t in range(1, NW):
            tot = tot + pbuf[pl.ds(t * NSEG + j * 16, 16)]
        invbuf[pl.ds(j * 16, 16)] = one16 / jnp.maximum(tot, one16)

    pltpu.sync_copy(b_hbm.at[pl.ds(p0, WROWS)], bbuf.at[pl.ds(0, WROWS)])
    pltpu.sync_copy(fl_hbm.at[pl.ds(p0, WROWS)], fbuf)

    def loop_body(j, _):
        sl = pl.ds(j * 16, 16)
        ids = jnp.clip(bbuf[sl], 0, NSEG - 1)
        iv = plsc.load_gather(invbuf, [ids])
        sbuf[sl] = fbuf[sl] * iv
        return 0
    lax.fori_loop(0, WROWS // 16, loop_body, 0)

    pltpu.sync_copy(sbuf, sc_hbm.at[pl.ds(p0, WROWS)])


_k1 = pl.kernel(
    _k1_body,
    out_type=jax.ShapeDtypeStruct((NW, EMBW), jnp.float32),
    mesh=_mesh(),
    compiler_params=pltpu.CompilerParams(needs_layout_passes=False),
    scratch_types=[
        pltpu.VMEM((2 * BRH,), jnp.float32),
        pltpu.VMEM((WROWS + 16,), jnp.int32),
        pltpu.VMEM((EMBW,), jnp.float32),
        pltpu.SemaphoreType.DMA,
        pltpu.SemaphoreType.DMA,
    ],
)

_k2 = pl.kernel(
    _k2_body,
    out_type=(
        jax.ShapeDtypeStruct((EMBW,), jnp.float32),
        jax.ShapeDtypeStruct((N_ROWS,), jnp.float32),
        jax.ShapeDtypeStruct((NW * NSEG,), jnp.float32),
    ),
    mesh=_mesh(),
    compiler_params=pltpu.CompilerParams(needs_layout_passes=False),
    scratch_types=[
        pltpu.VMEM((2 * BRH,), jnp.float32),
        pltpu.VMEM((WROWS + 16,), jnp.int32),
        pltpu.VMEM((EMBW,), jnp.float32),
        pltpu.VMEM((8 * SEGW,), jnp.float32),
        pltpu.VMEM((SEGW,), jnp.float32),
        pltpu.VMEM((NSEG,), jnp.float32),
        pltpu.VMEM((BR,), jnp.float32),
        pltpu.VMEM_SHARED((EMBW,), jnp.float32),
        pltpu.SemaphoreType.DMA,
        pltpu.SemaphoreType.DMA,
        pltpu.SemaphoreType.DMA,
    ],
)

_k3 = pl.kernel(
    _k3_body,
    out_type=jax.ShapeDtypeStruct((N_ROWS,), jnp.float32),
    mesh=_mesh(),
    compiler_params=pltpu.CompilerParams(needs_layout_passes=False),
    scratch_types=[
        pltpu.VMEM((NW * NSEG,), jnp.float32),
        pltpu.VMEM((NSEG,), jnp.float32),
        pltpu.VMEM((WROWS + 16,), jnp.int32),
        pltpu.VMEM((WROWS,), jnp.float32),
        pltpu.VMEM((WROWS,), jnp.float32),
    ],
)


def kernel(x, batch):
    xf = x.reshape(-1)
    part = _k1(xf, batch)
    emb, flags, pcnt = _k2(xf, batch, part)
    scores = _k3(batch, flags, pcnt)
    return emb.reshape(NSEG, HID), scores
